# Initial kernel scaffold; baseline (speedup 1.0000x reference)
import jax
import jax.numpy as jnp
from jax.experimental import pallas as pl


def kernel(x, edge_index, W1, b1, W2, b2):
    raise NotImplementedError("write your pallas kernel here")


# speedup vs baseline: 8.4862x; 8.4862x over previous
import functools

import jax
import jax.numpy as jnp
from jax import lax
from jax.experimental import pallas as pl
from jax.experimental.pallas import tpu as pltpu
from jax.experimental.pallas import tpu_sc as plsc

N = 10000
NP = 10240
E = 320000
NW = 32
CH = 128
CPW = 80
EPW = CH * CPW
EP = NW * EPW
RPT = NP // 16
RB = 1024

_mesh = plsc.VectorSubcoreMesh(core_axis_name="c", subcore_axis_name="s")


@functools.partial(
    pl.kernel,
    mesh=_mesh,
    out_type=jax.ShapeDtypeStruct((2, NP), jnp.float32),
    scratch_types=[
        pltpu.VMEM((CH,), jnp.int32),
        pltpu.VMEM((CH,), jnp.float32),
        pltpu.VMEM_SHARED((NP,), jnp.float32),
    ],
)
def _deg_kernel(dst_hbm, zeros_hbm, ones_hbm, out_hbm, idx_v, ones_v, deg_sh):
    cid = lax.axis_index("c")
    sid = lax.axis_index("s")
    wid = sid * 2 + cid
    row0 = sid * RPT
    pltpu.sync_copy(zeros_hbm.at[pl.ds(row0, RPT)], deg_sh.at[pl.ds(row0, RPT)])
    pltpu.sync_copy(ones_hbm, ones_v)
    plsc.subcore_barrier()
    base = wid * EPW

    def body(c, carry):
        off = base + c * CH
        pltpu.sync_copy(dst_hbm.at[pl.ds(off, CH)], idx_v)
        pltpu.sync_copy(ones_v, deg_sh.at[idx_v], add=True)
        return carry

    lax.fori_loop(0, CPW, body, 0)
    plsc.subcore_barrier()
    pltpu.sync_copy(deg_sh.at[pl.ds(row0, RPT)], out_hbm.at[cid, pl.ds(row0, RPT)])


def _make_agg(D):
    @functools.partial(
        pl.kernel,
        mesh=_mesh,
        compiler_params=pltpu.CompilerParams(use_tc_tiling_on_sc=False),
        out_type=jax.ShapeDtypeStruct((2, NP, D), jnp.float32),
        scratch_types=[
            pltpu.VMEM((CH,), jnp.int32),
            pltpu.VMEM((CH,), jnp.int32),
            pltpu.VMEM((CH, D), jnp.float32),
            pltpu.VMEM_SHARED((NP, D), jnp.float32),
            pltpu.SemaphoreType.DMA,
        ],
    )
    def agg(src_hbm, dst_hbm, xs_hbm, zeros_hbm, out_hbm, sidx, didx, buf, acc_sh, sem):
        cid = lax.axis_index("c")
        sid = lax.axis_index("s")
        wid = sid * 2 + cid
        row0 = sid * RPT
        pltpu.sync_copy(zeros_hbm.at[pl.ds(row0, RPT)], acc_sh.at[pl.ds(row0, RPT)])
        plsc.subcore_barrier()
        base = wid * EPW

        def body(c, carry):
            off = base + c * CH
            pltpu.sync_copy(src_hbm.at[pl.ds(off, CH)], sidx)
            pltpu.sync_copy(dst_hbm.at[pl.ds(off, CH)], didx)
            pltpu.async_copy(xs_hbm.at[sidx], buf, sem).wait()
            pltpu.sync_copy(buf, acc_sh.at[didx], add=True)
            return carry

        lax.fori_loop(0, CPW, body, 0)
        plsc.subcore_barrier()
        pltpu.sync_copy(acc_sh.at[pl.ds(row0, RPT)], out_hbm.at[cid, pl.ds(row0, RPT)])

    return agg


_agg128 = _make_agg(128)
_agg64 = _make_agg(64)


def _mm_scale(xp, dinvb, W, D):
    def body(x_ref, d_ref, w_ref, o_ref):
        o_ref[...] = jnp.dot(
            x_ref[...] * d_ref[...], w_ref[...], preferred_element_type=jnp.float32
        )

    return pl.pallas_call(
        body,
        grid=(NP // RB,),
        in_specs=[
            pl.BlockSpec((RB, 128), lambda i: (i, 0)),
            pl.BlockSpec((RB, 128), lambda i: (i, 0)),
            pl.BlockSpec((128, D), lambda i: (0, 0)),
        ],
        out_specs=pl.BlockSpec((RB, D), lambda i: (i, 0)),
        out_shape=jax.ShapeDtypeStruct((NP, D), jnp.float32),
    )(xp, dinvb, W)


def _layer1_combine(p0, p1, xs1, dinvb, b1, W2):
    def body(p0_ref, p1_ref, xs_ref, d_ref, b_ref, w_ref, z_ref, zs_ref):
        acc = p0_ref[...] + p1_ref[...] + xs_ref[...]
        z = jnp.maximum(d_ref[...] * acc + b_ref[...], 0.0)
        z_ref[...] = z
        zs_ref[...] = jnp.dot(
            d_ref[...] * z, w_ref[...], preferred_element_type=jnp.float32
        )

    return pl.pallas_call(
        body,
        grid=(NP // RB,),
        in_specs=[
            pl.BlockSpec((RB, 128), lambda i: (i, 0)),
            pl.BlockSpec((RB, 128), lambda i: (i, 0)),
            pl.BlockSpec((RB, 128), lambda i: (i, 0)),
            pl.BlockSpec((RB, 128), lambda i: (i, 0)),
            pl.BlockSpec((1, 128), lambda i: (0, 0)),
            pl.BlockSpec((128, 64), lambda i: (0, 0)),
        ],
        out_specs=[
            pl.BlockSpec((RB, 128), lambda i: (i, 0)),
            pl.BlockSpec((RB, 64), lambda i: (i, 0)),
        ],
        out_shape=[
            jax.ShapeDtypeStruct((NP, 128), jnp.float32),
            jax.ShapeDtypeStruct((NP, 64), jnp.float32),
        ],
    )(p0, p1, xs1, dinvb, b1, W2)


def _layer2_combine(q0, q1, zs2, dinvb, b2):
    def body(q0_ref, q1_ref, zs_ref, d_ref, b_ref, o_ref):
        acc = q0_ref[...] + q1_ref[...] + zs_ref[...]
        o_ref[...] = d_ref[..., :64] * acc + b_ref[...]

    return pl.pallas_call(
        body,
        grid=(NP // RB,),
        in_specs=[
            pl.BlockSpec((RB, 64), lambda i: (i, 0)),
            pl.BlockSpec((RB, 64), lambda i: (i, 0)),
            pl.BlockSpec((RB, 64), lambda i: (i, 0)),
            pl.BlockSpec((RB, 128), lambda i: (i, 0)),
            pl.BlockSpec((1, 64), lambda i: (0, 0)),
        ],
        out_specs=pl.BlockSpec((RB, 64), lambda i: (i, 0)),
        out_shape=jax.ShapeDtypeStruct((NP, 64), jnp.float32),
    )(q0, q1, zs2, dinvb, b2)


@jax.jit
def kernel(x, edge_index, W1, b1, W2, b2):
    src = edge_index[0].astype(jnp.int32)
    dst = edge_index[1].astype(jnp.int32)
    pad = jnp.full((EP - E,), N, dtype=jnp.int32)
    src_p = jnp.concatenate([src, pad])
    dst_p = jnp.concatenate([dst, pad])

    zeros1 = jnp.zeros((NP,), jnp.float32)
    ones_ch = jnp.ones((CH,), jnp.float32)
    degp = _deg_kernel(dst_p, zeros1, ones_ch)
    deg = degp[0] + degp[1] + 1.0
    dinv = lax.rsqrt(deg)
    dinvb = jnp.broadcast_to(dinv[:, None], (NP, 128))

    x_p = jnp.concatenate([x, jnp.zeros((NP - N, 128), jnp.float32)])
    xs1 = _mm_scale(x_p, dinvb, W1, 128)

    zeros128 = jnp.zeros((NP, 128), jnp.float32)
    p = _agg128(src_p, dst_p, xs1, zeros128)
    z_p, zs2 = _layer1_combine(p[0], p[1], xs1, dinvb, b1.reshape(1, 128), W2)

    zeros64 = jnp.zeros((NP, 64), jnp.float32)
    q = _agg64(src_p, dst_p, zs2, zeros64)
    logits_p = _layer2_combine(q[0], q[1], zs2, dinvb, b2.reshape(1, 64))

    return (logits_p[:N], z_p[:N])
